# baseline (speedup 1.0000x reference)
import jax
import jax.numpy as jnp
from jax import lax
from jax.experimental import pallas as pl
from jax.experimental.pallas import tpu as pltpu
from jax.experimental.pallas import tpu_sc as plsc

D_MODEL = 64
NUM_CORES = 2
NUM_SUBCORES = 16
NUM_WORKERS = NUM_CORES * NUM_SUBCORES
SEQ_PAD = 64
NBUF = 8
NSLICE = 4


def _make_body(slice_idx, tok_slice):
    def body(ids_hbm, table_hbm, out_hbm, idx_all, rows_v, gsems, osems):
        wid = lax.axis_index("s") * NUM_CORES + lax.axis_index("c")
        seq = out_hbm.shape[1]
        nb = idx_all.shape[0]
        base = slice_idx * tok_slice + wid * nb

        pltpu.sync_copy(ids_hbm.at[pl.ds(base, nb)], idx_all)

        def start_gather(b, k):
            pltpu.async_copy(
                table_hbm.at[idx_all.at[k]], rows_v.at[b], gsems.at[b]
            )

        def wait_gather(b, k):
            pltpu.make_async_copy(
                table_hbm.at[idx_all.at[k]], rows_v.at[b], gsems.at[b]
            ).wait()

        def store(b, k):
            pltpu.async_copy(
                rows_v.at[b, pl.ds(0, seq)],
                out_hbm.at[wid * nb + k],
                osems.at[b],
            )

        def wait_store(b, k):
            pltpu.make_async_copy(
                rows_v.at[b, pl.ds(0, seq)],
                out_hbm.at[wid * nb + k],
                osems.at[b],
            ).wait()

        for b in range(NBUF):
            start_gather(b, b)

        @pl.loop(0, nb, step=NBUF)
        def _(k0):
            for b in range(NBUF):
                k = k0 + b
                wait_gather(b, k)
                store(b, k)

                @pl.when(k + NBUF < nb)
                def _():
                    wait_store(b, k)
                    start_gather(b, k + NBUF)

        for b in range(NBUF):
            wait_store(b, nb - NBUF + b)

    return body


def kernel(token_ids, weight):
    n_tok, seq = token_ids.shape
    tok_slice = n_tok // NSLICE
    nb = tok_slice // NUM_WORKERS
    n_pad = SEQ_PAD - seq

    pad_vals = (
        jnp.arange(n_tok * n_pad, dtype=jnp.int32) % weight.shape[0]
    ).reshape(n_tok, n_pad)
    ids_pad = jnp.concatenate([token_ids.astype(jnp.int32), pad_vals], axis=1)

    mesh = plsc.VectorSubcoreMesh(core_axis_name="c", subcore_axis_name="s")
    outs = []
    for i in range(NSLICE):
        call = pl.kernel(
            _make_body(i, tok_slice),
            out_type=jax.ShapeDtypeStruct((tok_slice, seq, D_MODEL), jnp.float32),
            mesh=mesh,
            scratch_types=[
                pltpu.VMEM((nb, SEQ_PAD), jnp.int32),
                pltpu.VMEM((NBUF, SEQ_PAD, D_MODEL), jnp.float32),
                pltpu.SemaphoreType.DMA((NBUF,)),
                pltpu.SemaphoreType.DMA((NBUF,)),
            ],
            compiler_params=pltpu.CompilerParams(use_tc_tiling_on_sc=False),
        )
        outs.append(call(ids_pad, weight))
    return jnp.concatenate(outs, axis=0)


# speedup vs baseline: 1.0445x; 1.0445x over previous
import jax
import jax.numpy as jnp
from jax import lax
from jax.experimental import pallas as pl
from jax.experimental.pallas import tpu as pltpu
from jax.experimental.pallas import tpu_sc as plsc

D_MODEL = 64
NUM_CORES = 2
NUM_SUBCORES = 16
NUM_WORKERS = NUM_CORES * NUM_SUBCORES
CHUNK = 128
NBUF = 8


def _gather_body(ids_hbm, table_hbm, out_hbm, idx_all, rows_v, gsems, osems):
    wid = lax.axis_index("s") * NUM_CORES + lax.axis_index("c")
    b_per_w = idx_all.shape[0]
    nchunks = b_per_w // CHUNK
    base = wid * b_per_w

    pltpu.sync_copy(ids_hbm.at[pl.ds(pl.multiple_of(base, 8), b_per_w)], idx_all)

    def start_gather(b, g):
        idx = idx_all.at[pl.ds(g * CHUNK, CHUNK)]
        pltpu.async_copy(table_hbm.at[idx], rows_v.at[b], gsems.at[b])

    for b in range(NBUF):
        start_gather(b, b)

    @pl.loop(0, nchunks, step=NBUF)
    def _(g0):
        for b in range(NBUF):
            g = g0 + b
            idx = idx_all.at[pl.ds(g * CHUNK, CHUNK)]
            pltpu.make_async_copy(
                table_hbm.at[idx], rows_v.at[b], gsems.at[b]
            ).wait()
            off = pl.multiple_of(base + g * CHUNK, 8)
            out_slice = out_hbm.at[pl.ds(off, CHUNK)]
            pltpu.async_copy(rows_v.at[b], out_slice, osems.at[b])

            @pl.when(g + NBUF < nchunks)
            def _():
                pltpu.make_async_copy(rows_v.at[b], out_slice, osems.at[b]).wait()
                start_gather(b, g + NBUF)

    for b in range(NBUF):
        g = nchunks - NBUF + b
        off = pl.multiple_of(base + g * CHUNK, 8)
        pltpu.make_async_copy(
            rows_v.at[b], out_hbm.at[pl.ds(off, CHUNK)], osems.at[b]
        ).wait()


def kernel(token_ids, weight):
    n_tok, seq = token_ids.shape
    b_total = n_tok * seq
    b_per_w = b_total // NUM_WORKERS
    ids_flat = token_ids.reshape(b_total).astype(jnp.int32)

    mesh = plsc.VectorSubcoreMesh(core_axis_name="c", subcore_axis_name="s")
    out = pl.kernel(
        _gather_body,
        out_type=jax.ShapeDtypeStruct((b_total, D_MODEL), jnp.float32),
        mesh=mesh,
        scratch_types=[
            pltpu.VMEM((b_per_w,), jnp.int32),
            pltpu.VMEM((NBUF, CHUNK, D_MODEL), jnp.float32),
            pltpu.SemaphoreType.DMA((NBUF,)),
            pltpu.SemaphoreType.DMA((NBUF,)),
        ],
        compiler_params=pltpu.CompilerParams(use_tc_tiling_on_sc=False),
    )(ids_flat, weight)
    return out.reshape(n_tok, seq, D_MODEL)
